# Initial kernel scaffold; baseline (speedup 1.0000x reference)
import jax
import jax.numpy as jnp
from jax.experimental import pallas as pl


def kernel(t, c, n, Wt, Wc):
    raise NotImplementedError("write your pallas kernel here")


# speedup vs baseline: 1.7339x; 1.7339x over previous
import functools

import jax
import jax.numpy as jnp
import numpy as np
from jax import lax
from jax.experimental import pallas as pl
from jax.experimental.pallas import tpu as pltpu
from jax.experimental.pallas import tpu_sc as plsc

D = 50
V = 100000
B = 16384
K = 20

NC = 2
NS = 16
NW = NC * NS

CH = 128
TB = B // NW
NBR = B * K // NW


def _sc_gather(Wt, Wc, t, c, nf):
    mesh = plsc.VectorSubcoreMesh(core_axis_name="c", subcore_axis_name="s")

    @functools.partial(
        pl.kernel,
        mesh=mesh,
        compiler_params=pltpu.CompilerParams(use_tc_tiling_on_sc=False),
        out_type=(
            jax.ShapeDtypeStruct((B, D), jnp.float32),
            jax.ShapeDtypeStruct((B, D), jnp.float32),
            jax.ShapeDtypeStruct((B * K, D), jnp.float32),
        ),
        scratch_types=[
            pltpu.VMEM((CH,), jnp.int32),
            pltpu.VMEM((CH, D), jnp.float32),
            pltpu.SemaphoreType.DMA,
        ],
    )
    def k(wt_hbm, wc_hbm, t_hbm, c_hbm, n_hbm, vt_hbm, vc_hbm, vn_hbm,
          idx_v, rows_v, sem):
        wid = lax.axis_index("s") * NC + lax.axis_index("c")

        def gather_chunk(table, src_idx_hbm, dst_hbm, base):
            pltpu.sync_copy(src_idx_hbm.at[pl.ds(base, CH)], idx_v)
            pltpu.async_copy(table.at[idx_v], rows_v, sem).wait()
            pltpu.sync_copy(rows_v, dst_hbm.at[pl.ds(base, CH)])

        tbase = wid * TB

        def tc_body(j, carry):
            gather_chunk(wt_hbm, t_hbm, vt_hbm, tbase + j * CH)
            gather_chunk(wc_hbm, c_hbm, vc_hbm, tbase + j * CH)
            return carry

        lax.fori_loop(0, TB // CH, tc_body, 0)

        nbase = wid * NBR

        def n_body(j, carry):
            gather_chunk(wc_hbm, n_hbm, vn_hbm, nbase + j * CH)
            return carry

        lax.fori_loop(0, NBR // CH, n_body, 0)

    return k(Wt, Wc, t, c, nf)


_S = np.zeros((K * D, K), dtype=np.float32)
_S[np.arange(K * D), np.arange(K * D) // D] = 1.0

_BB = 128


def _tc_loss_body(vt_ref, vc_ref, vn_ref, s_ref, out_ref):
    i = pl.program_id(0)
    vt = vt_ref[...]
    vc = vc_ref[...]
    vn = vn_ref[...]
    s = s_ref[...]

    pos = jnp.sum(vt * vc, axis=1)
    vt_rep = jnp.concatenate([vt] * K, axis=1)
    prod = vn * vt_rep
    neg = lax.dot_general(prod, s, (((1,), (0,)), ((), ())),
                          precision=lax.Precision.HIGHEST,
                          preferred_element_type=jnp.float32)

    pos_l = -jnp.log(1.0 / (1.0 + jnp.exp(-pos)) + 1e-10)
    neg_l = -jnp.log(1.0 / (1.0 + jnp.exp(neg)) + 1e-10)
    partial = (jnp.sum(pos_l) + jnp.sum(neg_l)) * (1.0 / B)

    @pl.when(i == 0)
    def _():
        out_ref[0, 0] = jnp.float32(0.0)

    out_ref[0, 0] += partial


def _tc_loss(vt, vc, vn2):
    grid = (B // _BB,)
    return pl.pallas_call(
        _tc_loss_body,
        grid=grid,
        in_specs=[
            pl.BlockSpec((_BB, D), lambda i: (i, 0)),
            pl.BlockSpec((_BB, D), lambda i: (i, 0)),
            pl.BlockSpec((_BB, K * D), lambda i: (i, 0)),
            pl.BlockSpec((K * D, K), lambda i: (0, 0)),
        ],
        out_specs=pl.BlockSpec((1, 1), lambda i: (0, 0),
                               memory_space=pltpu.SMEM),
        out_shape=jax.ShapeDtypeStruct((1, 1), jnp.float32),
    )(vt, vc, vn2, jnp.asarray(_S))


def kernel(t, c, n, Wt, Wc):
    t = t.astype(jnp.int32)
    c = c.astype(jnp.int32)
    nf = n.reshape(-1).astype(jnp.int32)
    vt, vc, vn = _sc_gather(Wt, Wc, t, c, nf)
    vn2 = vn.reshape(B, K * D)
    loss = _tc_loss(vt, vc, vn2)
    return loss[0, 0]


# speedup vs baseline: 3.0139x; 1.7382x over previous
import functools

import jax
import jax.numpy as jnp
from jax import lax
from jax.experimental import pallas as pl
from jax.experimental.pallas import tpu as pltpu
from jax.experimental.pallas import tpu_sc as plsc

D = 50
V = 100000
B = 16384
K = 20

NC = 2
NS = 16
NW = NC * NS

CH = 128
TB = B // NW
NBR = B * K // NW
SUP = 16
RPC = SUP * K
NCHK = NBR // RPC
_SPLITS = ((0, 128), (128, 128), (256, 64))


def _fetch_n(wc_hbm, nix, buf, sem, s):
    hs = []
    for off, ln in _SPLITS:
        hs.append(pltpu.async_copy(
            wc_hbm.at[nix.at[pl.ds(s * RPC + off, ln)]],
            buf.at[pl.ds(off, ln)], sem))
    return hs


def _sc_scores(Wt, Wc, t, c, nf):
    mesh = plsc.VectorSubcoreMesh(core_axis_name="c", subcore_axis_name="s")

    @functools.partial(
        pl.kernel,
        mesh=mesh,
        compiler_params=pltpu.CompilerParams(use_tc_tiling_on_sc=False,
                                             needs_layout_passes=False),
        out_type=(
            jax.ShapeDtypeStruct((B,), jnp.float32),
            jax.ShapeDtypeStruct((B * K,), jnp.float32),
        ),
        scratch_types=[
            pltpu.VMEM((TB,), jnp.int32),
            pltpu.VMEM((TB,), jnp.int32),
            pltpu.VMEM((NBR,), jnp.int32),
            pltpu.VMEM((TB, D), jnp.float32),
            pltpu.VMEM((CH, D), jnp.float32),
            pltpu.VMEM((RPC, D), jnp.float32),
            pltpu.VMEM((RPC, D), jnp.float32),
            pltpu.VMEM((TB,), jnp.float32),
            pltpu.VMEM((NBR,), jnp.float32),
            pltpu.SemaphoreType.DMA,
            pltpu.SemaphoreType.DMA,
            pltpu.SemaphoreType.DMA,
        ],
    )
    def k(wt_hbm, wc_hbm, t_hbm, c_hbm, n_hbm, pos_hbm, neg_hbm,
          tix, cix, nix, vt_v, ctmp, nb0, nb1, pos_v, neg_v,
          sem0, semA, semB):
        wid = lax.axis_index("s") * NC + lax.axis_index("c")
        tb = wid * TB
        nb = wid * NBR

        pltpu.sync_copy(t_hbm.at[pl.ds(tb, TB)], tix)
        pltpu.sync_copy(c_hbm.at[pl.ds(tb, TB)], cix)
        pltpu.sync_copy(n_hbm.at[pl.ds(nb, NBR)], nix)

        hvt = [pltpu.async_copy(wt_hbm.at[tix.at[pl.ds(j * CH, CH)]],
                                vt_v.at[pl.ds(j * CH, CH)], sem0)
               for j in range(TB // CH)]
        hs = {0: _fetch_n(wc_hbm, nix, nb0, semA, 0)}

        iota = lax.iota(jnp.int32, 16)
        row20 = iota * K

        for h in hvt:
            h.wait()

        for j in range(TB // CH):
            pltpu.async_copy(wc_hbm.at[cix.at[pl.ds(j * CH, CH)]],
                             ctmp, sem0).wait()

            def pos_g(g, carry, _j=j):
                vtrow = iota + (_j * CH + g * 16)
                crow = iota + g * 16

                def pos_d(dd, acc):
                    dspl = jnp.full((16,), dd, jnp.int32)
                    a = plsc.load_gather(vt_v, [vtrow, dspl])
                    bb = plsc.load_gather(ctmp, [crow, dspl])
                    return acc + a * bb

                acc = lax.fori_loop(0, D, pos_d,
                                    jnp.zeros((16,), jnp.float32))
                plsc.store_scatter(pos_v, [vtrow], acc)
                return carry

            lax.fori_loop(0, CH // 16, pos_g, 0)

        bufs = (nb0, nb1)
        sems = (semA, semB)
        for s in range(NCHK):
            if s + 1 < NCHK:
                hs[s + 1] = _fetch_n(wc_hbm, nix, bufs[(s + 1) % 2],
                                     sems[(s + 1) % 2], s + 1)
            for h in hs.pop(s):
                h.wait()
            buf = bufs[s % 2]
            vtrow = iota + s * SUP

            def neg_d(dd, accs, _buf=buf, _vtrow=vtrow):
                dspl = jnp.full((16,), dd, jnp.int32)
                vtd = plsc.load_gather(vt_v, [_vtrow, dspl])
                out = []
                for kk in range(K):
                    vnv = plsc.load_gather(_buf, [row20 + kk, dspl])
                    out.append(accs[kk] + vnv * vtd)
                return tuple(out)

            accs = lax.fori_loop(
                0, D, neg_d,
                tuple(jnp.zeros((16,), jnp.float32) for _ in range(K)))
            for kk in range(K):
                plsc.store_scatter(neg_v, [row20 + (s * RPC + kk)],
                                   accs[kk])

        pltpu.sync_copy(pos_v, pos_hbm.at[pl.ds(tb, TB)])
        pltpu.sync_copy(neg_v, neg_hbm.at[pl.ds(nb, NBR)])

    return k(Wt, Wc, t, c, nf)


def _tc_loss_body(pos_ref, neg_ref, out_ref):
    pos = pos_ref[...]
    neg = neg_ref[...]
    pos_l = -jnp.log(1.0 / (1.0 + jnp.exp(-pos)) + 1e-10)
    neg_l = -jnp.log(1.0 / (1.0 + jnp.exp(neg)) + 1e-10)
    out_ref[0, 0] = (jnp.sum(pos_l) + jnp.sum(neg_l)) * (1.0 / B)


def _tc_loss(pos, neg):
    return pl.pallas_call(
        _tc_loss_body,
        out_specs=pl.BlockSpec(memory_space=pltpu.SMEM),
        out_shape=jax.ShapeDtypeStruct((1, 1), jnp.float32),
    )(pos.reshape(B // 128, 128), neg.reshape(B * K // 128, 128))


def kernel(t, c, n, Wt, Wc):
    t = t.astype(jnp.int32)
    c = c.astype(jnp.int32)
    nf = n.reshape(-1).astype(jnp.int32)
    pos, neg = _sc_scores(Wt, Wc, t, c, nf)
    return _tc_loss(pos, neg)[0, 0]
